# baseline (speedup 1.0000x reference)
import jax
import jax.numpy as jnp
from jax.experimental import pallas as pl

_B = 32
_C = 384
_G = 96
_GROUPBY = 4
_NSEL = 47


def _channel_mask():
    key = jax.random.key(42)
    keys = jax.random.split(key, _B)
    notp = jnp.arange(1, _G, dtype=jnp.int32)
    chosen = jax.vmap(lambda k: jax.random.permutation(k, notp)[:_NSEL])(keys)
    mask = jnp.zeros((_B, _G), jnp.float32).at[:, 0].set(1.0)
    mask = mask.at[jnp.arange(_B)[:, None], chosen].set(1.0)
    return jnp.repeat(mask, _GROUPBY, axis=1)


def _mul_body(x_ref, m_ref, o_ref):
    o_ref[...] = x_ref[...] * m_ref[...]


def kernel(input):
    B, C, H, W = input.shape
    hw = H * W
    m = _channel_mask().reshape(B * C, 1)
    x = input.reshape(B * C, hw)
    rows_per_block = 256
    out = pl.pallas_call(
        _mul_body,
        grid=(B * C // rows_per_block,),
        in_specs=[
            pl.BlockSpec((rows_per_block, hw), lambda i: (i, 0)),
            pl.BlockSpec((rows_per_block, 1), lambda i: (i, 0)),
        ],
        out_specs=pl.BlockSpec((rows_per_block, hw), lambda i: (i, 0)),
        out_shape=jax.ShapeDtypeStruct((B * C, hw), jnp.float32),
    )(x, m)
    return out.reshape(B, C, H, W)
